# Initial kernel scaffold; baseline (speedup 1.0000x reference)
import jax
import jax.numpy as jnp
from jax.experimental import pallas as pl


def kernel(x, parent, offset, num_graphs, axis):
    raise NotImplementedError("write your pallas kernel here")


# speedup vs baseline: 4.3924x; 4.3924x over previous
import functools

import jax
import jax.numpy as jnp
from jax import lax
from jax.experimental import pallas as pl
from jax.experimental.pallas import tpu as pltpu
from jax.experimental.pallas import tpu_sc as plsc

B = 16384
N = 32
NW = 32
GPW = B // NW
CH = 16
NCHUNK = GPW // CH

_TWO_OVER_PI = 0.6366197723675814
_MAGIC = 12582912.0
_P1 = 1.5703125
_P2 = 4.837512969970703125e-4
_P3 = 7.549789954891886e-8
_S1, _S2, _S3 = -1.6666654611e-1, 8.3321608736e-3, -1.9515295891e-4
_C1, _C2, _C3 = 4.166664568298827e-2, -1.388731625493765e-3, 2.443315711809948e-5


def _sincos(x):
    t = x * _TWO_OVER_PI
    kf = (t + _MAGIC) - _MAGIC
    ki = kf.astype(jnp.int32)
    r = x - kf * _P1
    r = r - kf * _P2
    r = r - kf * _P3
    z = r * r
    sp = r + r * z * (_S1 + z * (_S2 + z * _S3))
    cp = 1.0 - 0.5 * z + z * z * (_C1 + z * (_C2 + z * _C3))
    swap = (ki & 1) == 1
    s_b = jnp.where(swap, cp, sp)
    c_b = jnp.where(swap, sp, cp)
    s = jnp.where((ki & 2) == 2, -s_b, s_b)
    c = jnp.where(((ki + 1) & 2) == 2, -c_b, c_b)
    return c, s


def _bf16r(x):
    b = lax.bitcast_convert_type(x, jnp.int32)
    b = b + 32767 + ((b >> 16) & 1)
    b = b & jnp.int32(-65536)
    return lax.bitcast_convert_type(b, jnp.float32)


def _rnd9(m):
    return tuple(_bf16r(v) for v in m)


def _mm3(a, b):
    return tuple(
        a[3 * i + 0] * b[0 + j] + a[3 * i + 1] * b[3 + j] + a[3 * i + 2] * b[6 + j]
        for i in range(3) for j in range(3)
    )


def _sc_forward(x2, offs2, params):
    mesh = plsc.VectorSubcoreMesh(core_axis_name="c", subcore_axis_name="s")
    f32 = jnp.float32

    @functools.partial(
        pl.kernel,
        out_type=[
            jax.ShapeDtypeStruct((B, 3 * N), f32),
            jax.ShapeDtypeStruct((B, 9 * N), f32),
            jax.ShapeDtypeStruct((B, 3 * N), f32),
        ],
        mesh=mesh,
        compiler_params=pltpu.CompilerParams(needs_layout_passes=False),
        scratch_types=[
            pltpu.VMEM((12 * N, CH), f32),
            pltpu.VMEM((CH, N), f32),
            pltpu.VMEM((CH, 6 * N), f32),
            pltpu.VMEM((CH, 3 * N), f32),
            pltpu.VMEM((CH, 9 * N), f32),
            pltpu.VMEM((CH, 3 * N), f32),
        ],
    )
    def fk(x_hbm, offs_hbm, par_hbm, pos_hbm, rot_hbm, gpos_hbm,
           par_v, x_v, offs_v, pos_v, rot_v, gpos_v):
        cid = lax.axis_index("c")
        sid = lax.axis_index("s")
        wid = sid * 2 + cid
        pltpu.sync_copy(par_hbm, par_v)
        gvec = lax.iota(jnp.int32, 16)

        def fullv(v):
            return jnp.full((16,), v, jnp.int32)

        def param(n, j):
            return plsc.load_gather(par_v, [fullv(n * 12 + j), gvec])

        def local_mat(n):
            xv = plsc.load_gather(x_v, [gvec, fullv(n)])
            c6 = n * 6
            o = [plsc.load_gather(offs_v, [gvec, fullv(c6 + j)]) for j in range(6)]
            m11, m22, m33 = param(n, 0), param(n, 1), param(n, 2)
            m12, m13, m23 = param(n, 3), param(n, 4), param(n, 5)
            n1, n2, n3 = param(n, 6), param(n, 7), param(n, 8)
            anorm = param(n, 9)
            cth, sth = _sincos(xv * anorm)
            t0 = 1.0 - cth
            T = (cth + t0 * m11, t0 * m12 - sth * n3, t0 * m13 + sth * n2,
                 t0 * m12 + sth * n3, cth + t0 * m22, t0 * m23 - sth * n1,
                 t0 * m13 - sth * n2, t0 * m23 + sth * n1, cth + t0 * m33)
            cx, sx = _sincos(o[3])
            cy, sy = _sincos(o[4])
            cz, sz = _sincos(o[5])
            cx, sx = _bf16r(cx), _bf16r(sx)
            cy, sy = _bf16r(cy), _bf16r(sy)
            cz, sz = _bf16r(cz), _bf16r(sz)
            t00, t02 = _bf16r(cz * cy), _bf16r(cz * sy)
            t10, t12 = _bf16r(sz * cy), _bf16r(sz * sy)
            E = (t00, t02 * sx - sz * cx, t02 * cx + sz * sx,
                 t10, t12 * sx + cz * cx, t12 * cx - cz * sx,
                 -sy, cy * sx, cy * cx)
            L = _mm3(_rnd9(E), _rnd9(T))
            return L, (o[0], o[1], o[2])

        def store_node(n, rot, pos, gpos):
            for k in range(9):
                plsc.store_scatter(rot_v, [gvec, fullv(n * 9 + k)], rot[k])
            for k in range(3):
                plsc.store_scatter(pos_v, [gvec, fullv(n * 3 + k)], pos[k])
                plsc.store_scatter(gpos_v, [gvec, fullv(n * 3 + k)], gpos[k])

        def chunk_body(i, carry):
            g0 = wid * GPW + i * CH
            pltpu.sync_copy(x_hbm.at[pl.ds(g0, CH)], x_v)
            pltpu.sync_copy(offs_hbm.at[pl.ds(g0, CH)], offs_v)

            rot0, xyz0 = local_mat(jnp.int32(0))
            zero = jnp.zeros((16,), f32)
            store_node(jnp.int32(0), rot0, (zero, zero, zero), xyz0)

            def node_body(n, st):
                rot = st[0:9]
                pos = st[9:12]
                gps = st[12:15]
                L, xyz = local_mat(n)
                rp = _rnd9(rot)
                xr = tuple(_bf16r(v) for v in xyz)
                d = tuple(rp[3 * i + 0] * xr[0] + rp[3 * i + 1] * xr[1]
                          + rp[3 * i + 2] * xr[2] for i in range(3))
                pos = (pos[0] + d[0], pos[1] + d[1], pos[2] + d[2])
                gps = (gps[0] + d[0], gps[1] + d[1], gps[2] + d[2])
                rot = _mm3(rp, _rnd9(L))
                store_node(n, rot, pos, gps)
                return rot + pos + gps

            init = rot0 + (zero, zero, zero) + xyz0
            lax.fori_loop(1, N, node_body, init, unroll=False)

            pltpu.sync_copy(pos_v, pos_hbm.at[pl.ds(g0, CH)])
            pltpu.sync_copy(rot_v, rot_hbm.at[pl.ds(g0, CH)])
            pltpu.sync_copy(gpos_v, gpos_hbm.at[pl.ds(g0, CH)])
            return carry

        lax.fori_loop(0, NCHUNK, chunk_body, jnp.int32(0), unroll=False)

    return fk(x2, offs2, params)


def kernel(x, parent, offset, num_graphs, axis):
    f32 = jnp.float32
    x2 = x.reshape(B, N).astype(f32)
    offs2 = offset.reshape(B, 6 * N).astype(f32)
    ax0 = axis.reshape(B, N, 3)[0].astype(f32)
    anorm = jnp.sqrt(jnp.sum(ax0 * ax0, axis=-1))
    n1, n2, n3 = ax0[:, 0], ax0[:, 1], ax0[:, 2]
    params = jnp.stack(
        [n1 * n1, n2 * n2, n3 * n3, n1 * n2, n1 * n3, n2 * n3,
         n1, n2, n3, anorm, jnp.zeros_like(anorm), jnp.zeros_like(anorm)],
        axis=-1)
    params16 = jnp.broadcast_to(params.reshape(12 * N, 1), (12 * N, CH))
    params16 = jnp.asarray(params16, f32)

    pos2, rot2, gpos2 = _sc_forward(x2, offs2, params16)
    return (pos2.reshape(-1, 3), rot2.reshape(-1, 3, 3), gpos2.reshape(-1, 3))


# speedup vs baseline: 5.5192x; 1.2565x over previous
import functools

import jax
import jax.numpy as jnp
from jax import lax
from jax.experimental import pallas as pl
from jax.experimental.pallas import tpu as pltpu
from jax.experimental.pallas import tpu_sc as plsc

B = 16384
N = 32
NW = 32
GPW = B // NW
CH = 64
NLG = CH // 16
NCHUNK = GPW // CH

_TWO_OVER_PI = 0.6366197723675814
_MAGIC = 12582912.0
_P1 = 1.5703125
_P2 = 4.837512969970703125e-4
_P3 = 7.549789954891886e-8
_S1, _S2, _S3 = -1.6666654611e-1, 8.3321608736e-3, -1.9515295891e-4
_C1, _C2, _C3 = 4.166664568298827e-2, -1.388731625493765e-3, 2.443315711809948e-5


def _sincos(x):
    t = x * _TWO_OVER_PI
    kf = (t + _MAGIC) - _MAGIC
    ki = kf.astype(jnp.int32)
    r = x - kf * _P1
    r = r - kf * _P2
    r = r - kf * _P3
    z = r * r
    sp = r + r * z * (_S1 + z * (_S2 + z * _S3))
    cp = 1.0 - 0.5 * z + z * z * (_C1 + z * (_C2 + z * _C3))
    swap = (ki & 1) == 1
    s_b = jnp.where(swap, cp, sp)
    c_b = jnp.where(swap, sp, cp)
    s = jnp.where((ki & 2) == 2, -s_b, s_b)
    c = jnp.where(((ki + 1) & 2) == 2, -c_b, c_b)
    return c, s


def _bf16r(x):
    b = lax.bitcast_convert_type(x, jnp.int32)
    b = b + 32767 + ((b >> 16) & 1)
    b = b & jnp.int32(-65536)
    return lax.bitcast_convert_type(b, jnp.float32)


def _rnd9(m):
    return tuple(_bf16r(v) for v in m)


def _mm3(a, b):
    return tuple(
        a[3 * i + 0] * b[0 + j] + a[3 * i + 1] * b[3 + j] + a[3 * i + 2] * b[6 + j]
        for i in range(3) for j in range(3)
    )


def _sc_forward(x2, offs2, params):
    mesh = plsc.VectorSubcoreMesh(core_axis_name="c", subcore_axis_name="s")
    f32 = jnp.float32

    @functools.partial(
        pl.kernel,
        out_type=[
            jax.ShapeDtypeStruct((B * 3 * N // 128, 128), f32),
            jax.ShapeDtypeStruct((B * 9 * N // 128, 128), f32),
            jax.ShapeDtypeStruct((B * 3 * N // 128, 128), f32),
        ],
        mesh=mesh,
        compiler_params=pltpu.CompilerParams(needs_layout_passes=False),
        scratch_types=[
            pltpu.VMEM((12 * N * 16 // 128, 128), f32),
            pltpu.VMEM((CH * N // 128, 128), f32),
            pltpu.VMEM((CH * 6 * N // 128, 128), f32),
            pltpu.VMEM((CH * 3 * N // 128, 128), f32),
            pltpu.VMEM((CH * 9 * N // 128, 128), f32),
            pltpu.VMEM((CH * 3 * N // 128, 128), f32),
        ],
    )
    def fk(x_hbm, offs_hbm, par_hbm, pos_hbm, rot_hbm, gpos_hbm,
           par_v, x_v, offs_v, pos_v, rot_v, gpos_v):
        cid = lax.axis_index("c")
        sid = lax.axis_index("s")
        wid = sid * 2 + cid
        pltpu.sync_copy(par_hbm, par_v)
        gvec = lax.iota(jnp.int32, 16)

        def gather(ref, idx):
            return plsc.load_gather(ref, [idx >> 7, idx & 127])

        def scatter(ref, idx, v):
            plsc.store_scatter(ref, [idx >> 7, idx & 127], v)

        def param(n, j):
            return gather(par_v, (n * 12 + j) * 16 + gvec)

        def local_mat(n, xbase, obase):
            xv = gather(x_v, xbase + n)
            ob = obase + n * 6
            o = [gather(offs_v, ob + j) for j in range(6)]
            m11, m22, m33 = param(n, 0), param(n, 1), param(n, 2)
            m12, m13, m23 = param(n, 3), param(n, 4), param(n, 5)
            n1, n2, n3 = param(n, 6), param(n, 7), param(n, 8)
            anorm = param(n, 9)
            cth, sth = _sincos(xv * anorm)
            t0 = 1.0 - cth
            T = (cth + t0 * m11, t0 * m12 - sth * n3, t0 * m13 + sth * n2,
                 t0 * m12 + sth * n3, cth + t0 * m22, t0 * m23 - sth * n1,
                 t0 * m13 - sth * n2, t0 * m23 + sth * n1, cth + t0 * m33)
            cx, sx = _sincos(o[3])
            cy, sy = _sincos(o[4])
            cz, sz = _sincos(o[5])
            cx, sx = _bf16r(cx), _bf16r(sx)
            cy, sy = _bf16r(cy), _bf16r(sy)
            cz, sz = _bf16r(cz), _bf16r(sz)
            t00, t02 = _bf16r(cz * cy), _bf16r(cz * sy)
            t10, t12 = _bf16r(sz * cy), _bf16r(sz * sy)
            E = (t00, t02 * sx - sz * cx, t02 * cx + sz * sx,
                 t10, t12 * sx + cz * cx, t12 * cx - cz * sx,
                 -sy, cy * sx, cy * cx)
            L = _mm3(_rnd9(E), _rnd9(T))
            return L, (o[0], o[1], o[2])

        def store_node(n, pbase, rbase, rot, pos, gpos):
            for k in range(9):
                scatter(rot_v, rbase + n * 9 + k, rot[k])
            for k in range(3):
                scatter(pos_v, pbase + n * 3 + k, pos[k])
                scatter(gpos_v, pbase + n * 3 + k, gpos[k])

        def group_body(lg, carry):
            gv = lg * 16 + gvec
            xbase = gv * N
            obase = gv * (6 * N)
            pbase = gv * (3 * N)
            rbase = gv * (9 * N)

            rot0, xyz0 = local_mat(jnp.int32(0), xbase, obase)
            zero = jnp.zeros((16,), f32)
            store_node(jnp.int32(0), pbase, rbase, rot0, (zero, zero, zero), xyz0)

            def node_body(n, st):
                rot = st[0:9]
                pos = st[9:12]
                gps = st[12:15]
                L, xyz = local_mat(n, xbase, obase)
                rp = _rnd9(rot)
                xr = tuple(_bf16r(v) for v in xyz)
                d = tuple(rp[3 * i + 0] * xr[0] + rp[3 * i + 1] * xr[1]
                          + rp[3 * i + 2] * xr[2] for i in range(3))
                pos = (pos[0] + d[0], pos[1] + d[1], pos[2] + d[2])
                gps = (gps[0] + d[0], gps[1] + d[1], gps[2] + d[2])
                rot = _mm3(rp, _rnd9(L))
                store_node(n, pbase, rbase, rot, pos, gps)
                return rot + pos + gps

            init = rot0 + (zero, zero, zero) + xyz0
            lax.fori_loop(1, N, node_body, init, unroll=False)
            return carry

        def chunk_body(c, carry):
            xrow = wid * (GPW * N // 128) + c * (CH * N // 128)
            orow = wid * (GPW * 6 * N // 128) + c * (CH * 6 * N // 128)
            prow = wid * (GPW * 3 * N // 128) + c * (CH * 3 * N // 128)
            rrow = wid * (GPW * 9 * N // 128) + c * (CH * 9 * N // 128)
            pltpu.sync_copy(x_hbm.at[pl.ds(xrow, CH * N // 128)], x_v)
            pltpu.sync_copy(offs_hbm.at[pl.ds(orow, CH * 6 * N // 128)], offs_v)
            lax.fori_loop(0, NLG, group_body, jnp.int32(0), unroll=False)
            pltpu.sync_copy(pos_v, pos_hbm.at[pl.ds(prow, CH * 3 * N // 128)])
            pltpu.sync_copy(rot_v, rot_hbm.at[pl.ds(rrow, CH * 9 * N // 128)])
            pltpu.sync_copy(gpos_v, gpos_hbm.at[pl.ds(prow, CH * 3 * N // 128)])
            return carry

        lax.fori_loop(0, NCHUNK, chunk_body, jnp.int32(0), unroll=False)

    return fk(x2, offs2, params)


def kernel(x, parent, offset, num_graphs, axis):
    f32 = jnp.float32
    x2 = x.reshape(B * N // 128, 128).astype(f32)
    offs2 = offset.reshape(B * 6 * N // 128, 128).astype(f32)
    ax0 = axis.reshape(B, N, 3)[0].astype(f32)
    anorm = jnp.sqrt(jnp.sum(ax0 * ax0, axis=-1))
    n1, n2, n3 = ax0[:, 0], ax0[:, 1], ax0[:, 2]
    params = jnp.stack(
        [n1 * n1, n2 * n2, n3 * n3, n1 * n2, n1 * n3, n2 * n3,
         n1, n2, n3, anorm, jnp.zeros_like(anorm), jnp.zeros_like(anorm)],
        axis=-1)
    params16 = jnp.broadcast_to(params.reshape(12 * N, 1), (12 * N, 16))
    params16 = jnp.asarray(params16, f32).reshape(12 * N * 16 // 128, 128)

    pos2, rot2, gpos2 = _sc_forward(x2, offs2, params16)
    return (pos2.reshape(-1, 3), rot2.reshape(-1, 3, 3), gpos2.reshape(-1, 3))


# speedup vs baseline: 29.2201x; 5.2943x over previous
import functools

import jax
import jax.numpy as jnp
from jax import lax
from jax.experimental import pallas as pl
from jax.experimental.pallas import tpu as pltpu
from jax.experimental.pallas import tpu_sc as plsc

B = 16384
N = 32
NW = 32
GPW = B // NW
CH = 64
NLG = CH // 16
NCHUNK = GPW // CH
CC = CH * N

_TWO_OVER_PI = 0.6366197723675814
_MAGIC = 12582912.0
_P1 = 1.5703125
_P2 = 4.837512969970703125e-4
_P3 = 7.549789954891886e-8
_S1, _S2, _S3 = -1.6666654611e-1, 8.3321608736e-3, -1.9515295891e-4
_C1, _C2, _C3 = 4.166664568298827e-2, -1.388731625493765e-3, 2.443315711809948e-5


def _sincos(x):
    t = x * _TWO_OVER_PI
    kf = (t + _MAGIC) - _MAGIC
    ki = kf.astype(jnp.int32)
    r = x - kf * _P1
    r = r - kf * _P2
    r = r - kf * _P3
    z = r * r
    sp = r + r * z * (_S1 + z * (_S2 + z * _S3))
    cp = 1.0 - 0.5 * z + z * z * (_C1 + z * (_C2 + z * _C3))
    swap = (ki & 1) == 1
    s_b = jnp.where(swap, cp, sp)
    c_b = jnp.where(swap, sp, cp)
    s = jnp.where((ki & 2) == 2, -s_b, s_b)
    c = jnp.where(((ki + 1) & 2) == 2, -c_b, c_b)
    return c, s


def _bf16r(x):
    b = lax.bitcast_convert_type(x, jnp.int32)
    b = b + 32767 + ((b >> 16) & 1)
    b = b & jnp.int32(-65536)
    return lax.bitcast_convert_type(b, jnp.float32)


def _rnd9(m):
    return tuple(_bf16r(v) for v in m)


def _mm3(a, b):
    return tuple(
        a[3 * i + 0] * b[0 + j] + a[3 * i + 1] * b[3 + j] + a[3 * i + 2] * b[6 + j]
        for i in range(3) for j in range(3)
    )


def _sc_forward(xt, offst, params):
    mesh = plsc.VectorSubcoreMesh(core_axis_name="c", subcore_axis_name="s")
    f32 = jnp.float32

    @functools.partial(
        pl.kernel,
        out_type=[
            jax.ShapeDtypeStruct((3, B * N), f32),
            jax.ShapeDtypeStruct((3, 3, B * N), f32),
            jax.ShapeDtypeStruct((3, B * N), f32),
        ],
        mesh=mesh,
        compiler_params=pltpu.CompilerParams(needs_layout_passes=False,
                                             use_tc_tiling_on_sc=True),
        scratch_types=[
            pltpu.VMEM((12 * N * 16,), f32),
            pltpu.VMEM((1, CC), f32),
            pltpu.VMEM((6, CC), f32),
            pltpu.VMEM((3, CC), f32),
            pltpu.VMEM((3, 3, CC), f32),
            pltpu.VMEM((3, CC), f32),
        ],
    )
    def fk(x_hbm, offs_hbm, par_hbm, pos_hbm, rot_hbm, gpos_hbm,
           par_v, x_v, offs_v, pos_v, rot_v, gpos_v):
        cid = lax.axis_index("c")
        sid = lax.axis_index("s")
        wid = sid * 2 + cid
        pltpu.sync_copy(par_hbm, par_v)
        gvec = lax.iota(jnp.int32, 16)
        zvec = jnp.zeros((16,), jnp.int32)
        cvec = [jnp.full((16,), k, jnp.int32) for k in range(6)]

        def param(n, j):
            return plsc.load_gather(par_v, [(n * 12 + j) * 16 + gvec])

        def local_mat(n, colbase):
            col = colbase + n
            xv = plsc.load_gather(x_v, [zvec, col])
            o = [plsc.load_gather(offs_v, [cvec[j], col]) for j in range(6)]
            m11, m22, m33 = param(n, 0), param(n, 1), param(n, 2)
            m12, m13, m23 = param(n, 3), param(n, 4), param(n, 5)
            n1, n2, n3 = param(n, 6), param(n, 7), param(n, 8)
            anorm = param(n, 9)
            cth, sth = _sincos(xv * anorm)
            t0 = 1.0 - cth
            T = (cth + t0 * m11, t0 * m12 - sth * n3, t0 * m13 + sth * n2,
                 t0 * m12 + sth * n3, cth + t0 * m22, t0 * m23 - sth * n1,
                 t0 * m13 - sth * n2, t0 * m23 + sth * n1, cth + t0 * m33)
            cx, sx = _sincos(o[3])
            cy, sy = _sincos(o[4])
            cz, sz = _sincos(o[5])
            cx, sx = _bf16r(cx), _bf16r(sx)
            cy, sy = _bf16r(cy), _bf16r(sy)
            cz, sz = _bf16r(cz), _bf16r(sz)
            t00, t02 = _bf16r(cz * cy), _bf16r(cz * sy)
            t10, t12 = _bf16r(sz * cy), _bf16r(sz * sy)
            E = (t00, t02 * sx - sz * cx, t02 * cx + sz * sx,
                 t10, t12 * sx + cz * cx, t12 * cx - cz * sx,
                 -sy, cy * sx, cy * cx)
            L = _mm3(_rnd9(E), _rnd9(T))
            return L, (o[0], o[1], o[2]), col

        def store_node(col, rot, pos, gpos):
            for i in range(3):
                for j in range(3):
                    plsc.store_scatter(rot_v, [cvec[i], cvec[j], col], rot[3 * i + j])
            for k in range(3):
                plsc.store_scatter(pos_v, [cvec[k], col], pos[k])
                plsc.store_scatter(gpos_v, [cvec[k], col], gpos[k])

        def group_body(lg, carry):
            colbase = (lg * 16 + gvec) * N

            rot0, xyz0, col0 = local_mat(jnp.int32(0), colbase)
            zero = jnp.zeros((16,), f32)
            store_node(col0, rot0, (zero, zero, zero), xyz0)

            def node_body(n, st):
                rot = st[0:9]
                pos = st[9:12]
                gps = st[12:15]
                L, xyz, col = local_mat(n, colbase)
                rp = _rnd9(rot)
                xr = tuple(_bf16r(v) for v in xyz)
                d = tuple(rp[3 * i + 0] * xr[0] + rp[3 * i + 1] * xr[1]
                          + rp[3 * i + 2] * xr[2] for i in range(3))
                pos = (pos[0] + d[0], pos[1] + d[1], pos[2] + d[2])
                gps = (gps[0] + d[0], gps[1] + d[1], gps[2] + d[2])
                rot = _mm3(rp, _rnd9(L))
                store_node(col, rot, pos, gps)
                return rot + pos + gps

            init = rot0 + (zero, zero, zero) + xyz0
            lax.fori_loop(1, N, node_body, init, unroll=False)
            return carry

        def chunk_body(c, carry):
            c0 = (wid * GPW + c * CH) * N
            pltpu.sync_copy(x_hbm.at[:, pl.ds(c0, CC)], x_v)
            pltpu.sync_copy(offs_hbm.at[:, pl.ds(c0, CC)], offs_v)
            lax.fori_loop(0, NLG, group_body, jnp.int32(0), unroll=False)
            pltpu.sync_copy(pos_v, pos_hbm.at[:, pl.ds(c0, CC)])
            pltpu.sync_copy(rot_v, rot_hbm.at[:, :, pl.ds(c0, CC)])
            pltpu.sync_copy(gpos_v, gpos_hbm.at[:, pl.ds(c0, CC)])
            return carry

        lax.fori_loop(0, NCHUNK, chunk_body, jnp.int32(0), unroll=False)

    return fk(xt, offst, params)


def kernel(x, parent, offset, num_graphs, axis):
    f32 = jnp.float32
    xt = x.astype(f32).T
    offst = offset.astype(f32).T
    ax0 = axis.reshape(B, N, 3)[0].astype(f32)
    anorm = jnp.sqrt(jnp.sum(ax0 * ax0, axis=-1))
    n1, n2, n3 = ax0[:, 0], ax0[:, 1], ax0[:, 2]
    params = jnp.stack(
        [n1 * n1, n2 * n2, n3 * n3, n1 * n2, n1 * n3, n2 * n3,
         n1, n2, n3, anorm, jnp.zeros_like(anorm), jnp.zeros_like(anorm)],
        axis=-1)
    params16 = jnp.broadcast_to(params.reshape(12 * N, 1), (12 * N, 16))
    params16 = jnp.asarray(params16, f32).reshape(12 * N * 16)

    pos2, rot2, gpos2 = _sc_forward(xt, offst, params16)
    return (pos2.T, jnp.transpose(rot2, (2, 0, 1)), gpos2.T)
